# baseline (speedup 1.0000x reference)
import jax
import jax.numpy as jnp
from jax import lax
from jax.experimental import pallas as pl
from jax.experimental.pallas import tpu as pltpu
from jax.experimental.pallas import tpu_sc as plsc

N = 131072
D = 256
HD = D // 2
NSEG = 2048
NC = 2
NS = 16
L = 16
ROWS_T = N // NS
R = 64
NBLK = ROWS_T // R
GRPS = R // L
SH_ROWS = NSEG // NS
EPS = 1e-8
EPS2 = EPS * EPS


def _sc_body(f3d_hbm, f2d_hbm, sp_hbm, bi_hbm,
             acc_out, key_out, msp_out,
             ab_buf, sp0, sp1, bi0, bi1, ix0, ix1,
             stati, acc_sh,
             sem_in0, sem_in1, sem_sc0, sem_sc1):
    cid = lax.axis_index("c")
    sid = lax.axis_index("s")
    base = sid * ROWS_T
    col0 = cid * HD
    sps = (sp0, sp1)
    bis = (bi0, bi1)
    ixs = (ix0, ix1)
    sem_in = (sem_in0, sem_in1)
    sem_sc = (sem_sc0, sem_sc1)

    zv = jnp.zeros((L,), jnp.float32)
    for r in range(16):
        for j in range(D // L):
            ab_buf[0, r, pl.ds(j * L, L)] = zv
    for q in range(SH_ROWS // 16):
        dst = pl.ds(sid * SH_ROWS + q * 16, 16)
        pltpu.sync_copy(ab_buf.at[0, pl.ds(0, 16)], acc_sh.at[dst])

    stati[0] = jnp.zeros((L,), jnp.int32)
    stati[1] = jnp.zeros((L,), jnp.int32)

    plsc.subcore_barrier()

    def start_in(b, blk):
        rows = base + blk * R
        rt = base // 8 + blk * (R // 8)
        for q in range(R // 8):
            pltpu.async_copy(
                f3d_hbm.at[rt + q, cid],
                ab_buf.at[b, pl.ds(q * 8, 8), pl.ds(0, HD)], sem_in[b])
            pltpu.async_copy(
                f2d_hbm.at[rt + q, cid],
                ab_buf.at[b, pl.ds(q * 8, 8), pl.ds(HD, HD)], sem_in[b])
        pltpu.async_copy(sp_hbm.at[pl.ds(rows, R)], sps[b], sem_in[b])
        pltpu.async_copy(bi_hbm.at[pl.ds(rows, R)], bis[b], sem_in[b])

    def wait_in(b, blk):
        rows = base + blk * R
        rt = base // 8 + blk * (R // 8)
        for q in range(R // 8):
            pltpu.make_async_copy(
                f3d_hbm.at[rt + q, cid],
                ab_buf.at[b, pl.ds(q * 8, 8), pl.ds(0, HD)], sem_in[b]).wait()
            pltpu.make_async_copy(
                f2d_hbm.at[rt + q, cid],
                ab_buf.at[b, pl.ds(q * 8, 8), pl.ds(HD, HD)], sem_in[b]).wait()
        pltpu.make_async_copy(sp_hbm.at[pl.ds(rows, R)], sps[b], sem_in[b]).wait()
        pltpu.make_async_copy(bi_hbm.at[pl.ds(rows, R)], bis[b], sem_in[b]).wait()

    start_in(0, 0)
    start_in(1, 1)

    @pl.loop(0, NBLK, step=2)
    def _blocks(g):
        for b in (0, 1):
            blk = g + b
            rows = base + blk * R
            wait_in(b, blk)

            mk = stati[0]
            ms = stati[1]
            for v in range(GRPS):
                sl = pl.ds(v * L, L)
                spv = sps[b][sl]
                biv = bis[b][sl]
                ixs[b][sl] = spv + biv * 512
                mk = jnp.maximum(mk, biv * 1024 + spv)
                ms = jnp.maximum(ms, spv)
            stati[0] = mk
            stati[1] = ms

            pltpu.async_copy(ab_buf.at[b], acc_sh.at[ixs[b]], sem_sc[b], add=True)

            pltpu.make_async_copy(ab_buf.at[b], acc_sh.at[ixs[b]], sem_sc[b]).wait()

            @pl.when(blk + 2 < NBLK)
            def _refill():
                start_in(b, blk + 2)

    wid = sid * NC + cid
    pltpu.sync_copy(stati.at[0], key_out.at[wid])
    pltpu.sync_copy(stati.at[1], msp_out.at[wid])
    plsc.subcore_barrier()
    sh = pl.ds(sid * SH_ROWS, SH_ROWS)
    pltpu.sync_copy(acc_sh.at[sh], acc_out.at[cid, sh])


def _stage1(f3d, f2d, sp, bi):
    f32, i32 = jnp.float32, jnp.int32
    mesh = plsc.VectorSubcoreMesh(
        core_axis_name="c", subcore_axis_name="s", num_cores=NC, num_subcores=NS)
    return pl.kernel(
        _sc_body,
        out_type=(
            jax.ShapeDtypeStruct((NC, NSEG, D), f32),
            jax.ShapeDtypeStruct((NC * NS, L), i32),
            jax.ShapeDtypeStruct((NC * NS, L), i32),
        ),
        mesh=mesh,
        compiler_params=pltpu.CompilerParams(
            use_tc_tiling_on_sc=False, needs_layout_passes=False),
        scratch_types=(
            pltpu.VMEM((2, R, D), f32),
            pltpu.VMEM((R,), i32),
            pltpu.VMEM((R,), i32),
            pltpu.VMEM((R,), i32),
            pltpu.VMEM((R,), i32),
            pltpu.VMEM((R,), i32),
            pltpu.VMEM((R,), i32),
            pltpu.VMEM((2, L), i32),
            pltpu.VMEM_SHARED((NSEG, D), f32),
            pltpu.SemaphoreType.DMA,
            pltpu.SemaphoreType.DMA,
            pltpu.SemaphoreType.DMA,
            pltpu.SemaphoreType.DMA,
        ),
    )(f3d, f2d, sp, bi)


BKP = 2048


def _tc_point_body(a_ref, b_ref, out_ref):
    a = a_ref[...]
    bm = b_ref[...]
    d = jnp.sum(a * bm, axis=1)
    na = jnp.maximum(jnp.sqrt(jnp.sum(a * a, axis=1)), EPS)
    nb = jnp.maximum(jnp.sqrt(jnp.sum(bm * bm, axis=1)), EPS)
    out_ref[0, 0, 0] = jnp.sum(d / (na * nb))


_tc_point = pl.pallas_call(
    _tc_point_body,
    grid=(N // BKP,),
    in_specs=[pl.BlockSpec((BKP, D), lambda i: (i, 0)),
              pl.BlockSpec((BKP, D), lambda i: (i, 0))],
    out_specs=pl.BlockSpec((1, 1, 1), lambda i: (i, 0, 0), memory_space=pltpu.SMEM),
    out_shape=jax.ShapeDtypeStruct((N // BKP, 1, 1), jnp.float32))


def _tc_body(acc_ref, simp_ref, key_ref, msp_ref, out_ref):
    sum_sim_p = jnp.sum(simp_ref[...])

    a0 = acc_ref[0, :, :HD]
    b0 = acc_ref[0, :, HD:]
    a1 = acc_ref[1, :, :HD]
    b1 = acc_ref[1, :, HD:]
    dots = jnp.sum(a0 * b0, axis=1) + jnp.sum(a1 * b1, axis=1)
    sa2 = jnp.sum(a0 * a0, axis=1) + jnp.sum(a1 * a1, axis=1)
    sb2 = jnp.sum(b0 * b0, axis=1) + jnp.sum(b1 * b1, axis=1)
    sna = jnp.maximum(jnp.sqrt(sa2), EPS)
    snb = jnp.maximum(jnp.sqrt(sb2), EPS)
    sum_sim_sp = jnp.sum(dots / (sna * snb))

    maxkey = jnp.max(key_ref[...])
    maxsp = jnp.max(msp_ref[...])
    bstar = maxkey // 1024
    m2 = maxkey - bstar * 1024
    nseg = (bstar * (maxsp + 1) + m2 + 1).astype(jnp.float32)
    out_ref[0, 0] = 2.0 - sum_sim_p * (1.0 / N) - sum_sim_sp / nseg


_tc_stage = pl.pallas_call(
    _tc_body,
    out_shape=jax.ShapeDtypeStruct((1, 1), jnp.float32),
    out_specs=pl.BlockSpec(memory_space=pltpu.SMEM))


def kernel(F3D, Fraw2D, superpoint_ids, batch_idx):
    f3d4 = F3D.reshape(N // 8, 8, NC, HD).transpose(0, 2, 1, 3)
    f2d4 = Fraw2D.reshape(N // 8, 8, NC, HD).transpose(0, 2, 1, 3)
    acc, keys, msps = _stage1(
        f3d4, f2d4,
        superpoint_ids.astype(jnp.int32), batch_idx.astype(jnp.int32))
    simp = _tc_point(F3D, Fraw2D)
    return _tc_stage(acc, simp, keys, msps)[0, 0]


# speedup vs baseline: 1.0347x; 1.0347x over previous
import jax
import jax.numpy as jnp
from jax import lax
from jax.experimental import pallas as pl
from jax.experimental.pallas import tpu as pltpu
from jax.experimental.pallas import tpu_sc as plsc

N = 131072
D = 256
HD = D // 2
NSEG = 2048
NC = 2
NS = 16
L = 16
ROWS_T = N // NS
R = 64
NBLK = ROWS_T // R
GRPS = R // L
SH_ROWS = NSEG // NS
EPS = 1e-8


def _sc_body(f3d_hbm, f2d_hbm, sp_hbm, bi_hbm,
             acc_a_out, acc_b_out, key_out, msp_out,
             a_buf, b_buf, sp0, sp1, bi0, bi1, ix0, ix1,
             stati, acc_a_sh, acc_b_sh,
             sem_in0, sem_in1, sem_sc0, sem_sc1):
    cid = lax.axis_index("c")
    sid = lax.axis_index("s")
    base = sid * ROWS_T
    col0 = cid * HD
    sps = (sp0, sp1)
    bis = (bi0, bi1)
    ixs = (ix0, ix1)
    sem_in = (sem_in0, sem_in1)
    sem_sc = (sem_sc0, sem_sc1)

    zv = jnp.zeros((L,), jnp.float32)
    for r in range(16):
        for j in range(HD // L):
            a_buf[0, r, pl.ds(j * L, L)] = zv
    for q in range(SH_ROWS // 16):
        dst = pl.ds(sid * SH_ROWS + q * 16, 16)
        pltpu.sync_copy(a_buf.at[0, pl.ds(0, 16)], acc_a_sh.at[dst])
        pltpu.sync_copy(a_buf.at[0, pl.ds(0, 16)], acc_b_sh.at[dst])

    stati[0] = jnp.zeros((L,), jnp.int32)
    stati[1] = jnp.zeros((L,), jnp.int32)

    plsc.subcore_barrier()

    def start_in(b, blk):
        rows = base + blk * R
        rt = base // 8 + blk * (R // 8)
        for q in range(R // 8):
            pltpu.async_copy(
                f3d_hbm.at[rt + q, cid], a_buf.at[b, pl.ds(q * 8, 8)], sem_in[b])
            pltpu.async_copy(
                f2d_hbm.at[rt + q, cid], b_buf.at[b, pl.ds(q * 8, 8)], sem_in[b])
        pltpu.async_copy(sp_hbm.at[pl.ds(rows, R)], sps[b], sem_in[b])
        pltpu.async_copy(bi_hbm.at[pl.ds(rows, R)], bis[b], sem_in[b])

    def wait_in(b, blk):
        rows = base + blk * R
        rt = base // 8 + blk * (R // 8)
        for q in range(R // 8):
            pltpu.make_async_copy(
                f3d_hbm.at[rt + q, cid], a_buf.at[b, pl.ds(q * 8, 8)], sem_in[b]).wait()
            pltpu.make_async_copy(
                f2d_hbm.at[rt + q, cid], b_buf.at[b, pl.ds(q * 8, 8)], sem_in[b]).wait()
        pltpu.make_async_copy(sp_hbm.at[pl.ds(rows, R)], sps[b], sem_in[b]).wait()
        pltpu.make_async_copy(bi_hbm.at[pl.ds(rows, R)], bis[b], sem_in[b]).wait()

    start_in(0, 0)
    start_in(1, 1)

    @pl.loop(0, NBLK, step=2)
    def _blocks(g):
        for b in (0, 1):
            blk = g + b
            rows = base + blk * R
            wait_in(b, blk)

            mk = stati[0]
            ms = stati[1]
            for v in range(GRPS):
                sl = pl.ds(v * L, L)
                spv = sps[b][sl]
                biv = bis[b][sl]
                ixs[b][sl] = spv + biv * 512
                mk = jnp.maximum(mk, biv * 1024 + spv)
                ms = jnp.maximum(ms, spv)
            stati[0] = mk
            stati[1] = ms

            pltpu.async_copy(a_buf.at[b], acc_a_sh.at[ixs[b]], sem_sc[b], add=True)
            pltpu.async_copy(b_buf.at[b], acc_b_sh.at[ixs[b]], sem_sc[b], add=True)

            pltpu.make_async_copy(a_buf.at[b], acc_a_sh.at[ixs[b]], sem_sc[b]).wait()
            pltpu.make_async_copy(b_buf.at[b], acc_b_sh.at[ixs[b]], sem_sc[b]).wait()

            @pl.when(blk + 2 < NBLK)
            def _refill():
                start_in(b, blk + 2)

    wid = sid * NC + cid
    pltpu.sync_copy(stati.at[0], key_out.at[wid])
    pltpu.sync_copy(stati.at[1], msp_out.at[wid])
    plsc.subcore_barrier()
    sh = pl.ds(sid * SH_ROWS, SH_ROWS)
    pltpu.sync_copy(acc_a_sh.at[sh], acc_a_out.at[cid, sh])
    pltpu.sync_copy(acc_b_sh.at[sh], acc_b_out.at[cid, sh])


def _stage1(f3d, f2d, sp, bi):
    f32, i32 = jnp.float32, jnp.int32
    mesh = plsc.VectorSubcoreMesh(
        core_axis_name="c", subcore_axis_name="s", num_cores=NC, num_subcores=NS)
    return pl.kernel(
        _sc_body,
        out_type=(
            jax.ShapeDtypeStruct((NC, NSEG, HD), f32),
            jax.ShapeDtypeStruct((NC, NSEG, HD), f32),
            jax.ShapeDtypeStruct((NC * NS, L), i32),
            jax.ShapeDtypeStruct((NC * NS, L), i32),
        ),
        mesh=mesh,
        compiler_params=pltpu.CompilerParams(
            use_tc_tiling_on_sc=False, needs_layout_passes=False),
        scratch_types=(
            pltpu.VMEM((2, R, HD), f32),
            pltpu.VMEM((2, R, HD), f32),
            pltpu.VMEM((R,), i32),
            pltpu.VMEM((R,), i32),
            pltpu.VMEM((R,), i32),
            pltpu.VMEM((R,), i32),
            pltpu.VMEM((R,), i32),
            pltpu.VMEM((R,), i32),
            pltpu.VMEM((2, L), i32),
            pltpu.VMEM_SHARED((NSEG, HD), f32),
            pltpu.VMEM_SHARED((NSEG, HD), f32),
            pltpu.SemaphoreType.DMA,
            pltpu.SemaphoreType.DMA,
            pltpu.SemaphoreType.DMA,
            pltpu.SemaphoreType.DMA,
        ),
    )(f3d, f2d, sp, bi)


BKP = 2048


def _tc_point_body(a_ref, b_ref, out_ref):
    a = a_ref[...]
    bm = b_ref[...]
    d = jnp.sum(a * bm, axis=1)
    na = jnp.maximum(jnp.sqrt(jnp.sum(a * a, axis=1)), EPS)
    nb = jnp.maximum(jnp.sqrt(jnp.sum(bm * bm, axis=1)), EPS)
    out_ref[0, 0, 0] = jnp.sum(d / (na * nb))


_tc_point = pl.pallas_call(
    _tc_point_body,
    grid=(N // BKP,),
    in_specs=[pl.BlockSpec((BKP, D), lambda i: (i, 0)),
              pl.BlockSpec((BKP, D), lambda i: (i, 0))],
    out_specs=pl.BlockSpec((1, 1, 1), lambda i: (i, 0, 0), memory_space=pltpu.SMEM),
    out_shape=jax.ShapeDtypeStruct((N // BKP, 1, 1), jnp.float32))


def _tc_body(acc_a_ref, acc_b_ref, simp_ref, key_ref, msp_ref, out_ref):
    sum_sim_p = jnp.sum(simp_ref[...])

    dots = (jnp.sum(acc_a_ref[0] * acc_b_ref[0], axis=1)
            + jnp.sum(acc_a_ref[1] * acc_b_ref[1], axis=1))
    sa2 = (jnp.sum(acc_a_ref[0] * acc_a_ref[0], axis=1)
           + jnp.sum(acc_a_ref[1] * acc_a_ref[1], axis=1))
    sb2 = (jnp.sum(acc_b_ref[0] * acc_b_ref[0], axis=1)
           + jnp.sum(acc_b_ref[1] * acc_b_ref[1], axis=1))
    sna = jnp.maximum(jnp.sqrt(sa2), EPS)
    snb = jnp.maximum(jnp.sqrt(sb2), EPS)
    sum_sim_sp = jnp.sum(dots / (sna * snb))

    maxkey = jnp.max(key_ref[...])
    maxsp = jnp.max(msp_ref[...])
    bstar = maxkey // 1024
    m2 = maxkey - bstar * 1024
    nseg = (bstar * (maxsp + 1) + m2 + 1).astype(jnp.float32)
    out_ref[0, 0] = 2.0 - sum_sim_p * (1.0 / N) - sum_sim_sp / nseg


_tc_stage = pl.pallas_call(
    _tc_body,
    out_shape=jax.ShapeDtypeStruct((1, 1), jnp.float32),
    out_specs=pl.BlockSpec(memory_space=pltpu.SMEM))


def kernel(F3D, Fraw2D, superpoint_ids, batch_idx):
    f3d4 = F3D.reshape(N // 8, 8, NC, HD).transpose(0, 2, 1, 3)
    f2d4 = Fraw2D.reshape(N // 8, 8, NC, HD).transpose(0, 2, 1, 3)
    acc_a, acc_b, keys, msps = _stage1(
        f3d4, f2d4,
        superpoint_ids.astype(jnp.int32), batch_idx.astype(jnp.int32))
    simp = _tc_point(F3D, Fraw2D)
    return _tc_stage(acc_a, acc_b, simp, keys, msps)[0, 0]
